# baseline (speedup 1.0000x reference)
import functools

import jax
import jax.numpy as jnp
from jax import lax
from jax.experimental import pallas as pl
from jax.experimental.pallas import tpu as pltpu
from jax.experimental.pallas import tpu_sc as plsc

N = 10000
E = 320000
D = 128
NC = 2
NS = 16
NW = NC * NS
CK = 128
EP = 327680
PAD = EP - E
EPW = EP // NW
NCHUNK = EPW // CK
HCHUNK = NCHUNK // 2
NP = 10240
SLICE1 = NP // NS
ROWS_PT = NP // NS
ZROWS = 128
BR = 1000

_MESH = plsc.VectorSubcoreMesh(core_axis_name="c", subcore_axis_name="s")


@functools.partial(
    pl.kernel,
    out_type=jax.ShapeDtypeStruct((NC, NP), jnp.float32),
    mesh=_MESH,
    compiler_params=pltpu.CompilerParams(needs_layout_passes=False),
    scratch_types=[
        pltpu.VMEM((NCHUNK, CK), jnp.int32),
        pltpu.VMEM((CK,), jnp.float32),
        pltpu.VMEM((SLICE1,), jnp.float32),
        pltpu.VMEM_SHARED((NP,), jnp.float32),
        pltpu.SemaphoreType.DMA,
    ],
)
def _sc_deg(col3_hbm, deg_hbm, col_v, ones_v, zb_v, acc_sh, dsem):
    cid = lax.axis_index("c")
    sid = lax.axis_index("s")
    wid = cid * NS + sid
    for j in range(SLICE1 // 16):
        zb_v[pl.ds(j * 16, 16)] = jnp.zeros((16,), jnp.float32)
    for j in range(CK // 16):
        ones_v[pl.ds(j * 16, 16)] = jnp.ones((16,), jnp.float32)
    off = pl.multiple_of(sid * SLICE1, 8)
    pltpu.sync_copy(zb_v, acc_sh.at[pl.ds(off, SLICE1)])
    plsc.subcore_barrier()
    pltpu.sync_copy(col3_hbm.at[wid], col_v)

    def adesc(i):
        return pltpu.make_async_copy(ones_v, acc_sh.at[col_v.at[i]], dsem)

    def body(i, carry):
        a = i * 4
        for k in range(4):
            adesc(a + k).start(add=True)
        for k in range(4):
            adesc(a + k).wait()
        return carry

    lax.fori_loop(0, NCHUNK // 4, body, 0)
    plsc.subcore_barrier()
    pltpu.sync_copy(acc_sh.at[pl.ds(off, SLICE1)],
                    deg_hbm.at[cid, pl.ds(off, SLICE1)])


@functools.partial(
    pl.kernel,
    out_type=jax.ShapeDtypeStruct((NC, NP, D), jnp.float32),
    mesh=_MESH,
    compiler_params=pltpu.CompilerParams(needs_layout_passes=False),
    scratch_types=[
        pltpu.VMEM((HCHUNK, CK), jnp.int32),
        pltpu.VMEM((HCHUNK, CK), jnp.int32),
        pltpu.VMEM((CK, D), jnp.float32),
        pltpu.VMEM((CK, D), jnp.float32),
        pltpu.VMEM_SHARED((NP, D), jnp.float32),
        pltpu.SemaphoreType.DMA,
        pltpu.SemaphoreType.DMA,
    ],
)
def _sc_agg1(g_hbm, row3_hbm, col3_hbm, agg_hbm,
             row_v, col_v, rbuf0, rbuf1, acc_sh, gsem, ssem):
    cid = lax.axis_index("c")
    sid = lax.axis_index("s")
    wid = cid * NS + sid

    def zbody(r, carry):
        for c in range(D // 16):
            rbuf0[r, pl.ds(c * 16, 16)] = jnp.zeros((16,), jnp.float32)
        return carry

    lax.fori_loop(0, CK, zbody, 0)
    base = pl.multiple_of(sid * ROWS_PT, 8)
    for j in range(ROWS_PT // CK):
        pltpu.sync_copy(rbuf0, acc_sh.at[pl.ds(base + j * CK, CK)])
    plsc.subcore_barrier()

    def gdesc(i, buf):
        return pltpu.make_async_copy(g_hbm.at[row_v.at[i]], buf, gsem)

    def sdesc(i, buf):
        return pltpu.make_async_copy(buf, acc_sh.at[col_v.at[i]], ssem)

    for half in range(2):
        pltpu.sync_copy(row3_hbm.at[wid, pl.ds(half * HCHUNK, HCHUNK)], row_v)
        pltpu.sync_copy(col3_hbm.at[wid, pl.ds(half * HCHUNK, HCHUNK)], col_v)
        gdesc(0, rbuf0).start()
        gdesc(1, rbuf1).start()

        def body(i, carry):
            a = i * 2
            gdesc(a, rbuf0).wait()
            sdesc(a, rbuf0).start(add=True)
            gdesc(a + 1, rbuf1).wait()
            sdesc(a, rbuf0).wait()

            @pl.when(a + 2 < HCHUNK)
            def _():
                gdesc(a + 2, rbuf0).start()

            sdesc(a + 1, rbuf1).start(add=True)
            sdesc(a + 1, rbuf1).wait()

            @pl.when(a + 3 < HCHUNK)
            def _():
                gdesc(a + 3, rbuf1).start()

            return carry

        lax.fori_loop(0, HCHUNK // 2, body, 0)
    plsc.subcore_barrier()
    pltpu.sync_copy(acc_sh.at[pl.ds(base, ROWS_PT)],
                    agg_hbm.at[cid, pl.ds(base, ROWS_PT)])


@functools.partial(
    pl.kernel,
    out_type=jax.ShapeDtypeStruct((NC, NP), jnp.float32),
    mesh=_MESH,
    compiler_params=pltpu.CompilerParams(needs_layout_passes=False),
    scratch_types=[
        pltpu.VMEM((NCHUNK, CK), jnp.int32),
        pltpu.VMEM((NCHUNK, CK), jnp.int32),
        pltpu.VMEM((N,), jnp.float32),
        pltpu.VMEM((16,), jnp.float32),
        pltpu.VMEM((CK,), jnp.float32),
        pltpu.VMEM((CK,), jnp.float32),
        pltpu.VMEM((SLICE1,), jnp.float32),
        pltpu.VMEM((2 * SLICE1,), jnp.float32),
        pltpu.VMEM_SHARED((NP,), jnp.float32),
        pltpu.SemaphoreType.DMA,
    ],
)
def _sc_agg2(q_hbm, row3_hbm, col3_hbm, dis_hbm, b2_hbm, out_hbm,
             row_v, col_v, q_v, b2_v, vals0, vals1, zb_v, dq_v, acc_sh, asem):
    cid = lax.axis_index("c")
    sid = lax.axis_index("s")
    wid = cid * NS + sid
    for j in range(SLICE1 // 16):
        zb_v[pl.ds(j * 16, 16)] = jnp.zeros((16,), jnp.float32)
    off = pl.multiple_of(sid * SLICE1, 8)
    pltpu.sync_copy(zb_v, acc_sh.at[pl.ds(off, SLICE1)])
    plsc.subcore_barrier()
    pltpu.sync_copy(q_hbm.at[pl.ds(0, N)], q_v)
    pltpu.sync_copy(row3_hbm.at[wid], row_v)
    pltpu.sync_copy(col3_hbm.at[wid], col_v)
    pltpu.sync_copy(b2_hbm, b2_v)

    def fill(i, buf):
        for j in range(CK // 16):
            idx16 = row_v[i, pl.ds(j * 16, 16)]
            buf[pl.ds(j * 16, 16)] = plsc.load_gather(q_v, [idx16])

    def adesc(i, buf):
        return pltpu.make_async_copy(buf, acc_sh.at[col_v.at[i]], asem)

    def body(i, carry):
        a = i * 2
        fill(a, vals0)
        adesc(a, vals0).start(add=True)
        fill(a + 1, vals1)
        adesc(a + 1, vals1).start(add=True)
        adesc(a, vals0).wait()
        adesc(a + 1, vals1).wait()
        return carry

    lax.fori_loop(0, NCHUNK // 2, body, 0)
    plsc.subcore_barrier()
    pltpu.sync_copy(dis_hbm.at[pl.ds(off, SLICE1)], dq_v.at[pl.ds(0, SLICE1)])
    pltpu.sync_copy(q_hbm.at[pl.ds(off, SLICE1)], dq_v.at[pl.ds(SLICE1, SLICE1)])
    pltpu.sync_copy(acc_sh.at[pl.ds(off, SLICE1)], zb_v)
    for j in range(SLICE1 // 16):
        d16 = dq_v[pl.ds(j * 16, 16)]
        q16 = dq_v[pl.ds(SLICE1 + j * 16, 16)]
        a16 = zb_v[pl.ds(j * 16, 16)]
        zb_v[pl.ds(j * 16, 16)] = d16 * (a16 + 0.5 * q16) + 0.5 * b2_v[...]
    pltpu.sync_copy(zb_v, out_hbm.at[cid, pl.ds(off, SLICE1)])


def _tc_h_body(x_ref, w1_ref, h_ref):
    h_ref[...] = jnp.dot(x_ref[...], w1_ref[...],
                         preferred_element_type=jnp.float32)


_tc_h = pl.pallas_call(
    _tc_h_body,
    grid=(N // BR,),
    in_specs=[
        pl.BlockSpec((BR, D), lambda i: (i, 0)),
        pl.BlockSpec((D, D), lambda i: (0, 0)),
    ],
    out_specs=pl.BlockSpec((BR, D), lambda i: (i, 0)),
    out_shape=jax.ShapeDtypeStruct((N, D), jnp.float32),
)


def _tc_a_body(h_ref, d0_ref, d1_ref, g_ref, dis_ref):
    deg = d0_ref[...] + d1_ref[...] + 1.0
    dis = lax.rsqrt(deg)
    g_ref[...] = h_ref[...] * dis
    dis_ref[...] = dis


_tc_a = pl.pallas_call(
    _tc_a_body,
    grid=(N // BR,),
    in_specs=[
        pl.BlockSpec((BR, D), lambda i: (i, 0)),
        pl.BlockSpec((BR, 1), lambda i: (i, 0)),
        pl.BlockSpec((BR, 1), lambda i: (i, 0)),
    ],
    out_specs=[
        pl.BlockSpec((BR, D), lambda i: (i, 0)),
        pl.BlockSpec((BR, 1), lambda i: (i, 0)),
    ],
    out_shape=[
        jax.ShapeDtypeStruct((N, D), jnp.float32),
        jax.ShapeDtypeStruct((N, 1), jnp.float32),
    ],
)


def _tc_b_body(ap_ref, g_ref, dis_ref, b1_ref, w2_ref, q_ref):
    s = ap_ref[0] + ap_ref[1] + g_ref[...]
    out1 = dis_ref[...] * s + b1_ref[...]
    h1 = jnp.maximum(out1, 0.0)
    p = jnp.dot(h1, w2_ref[...], preferred_element_type=jnp.float32)
    q_ref[...] = dis_ref[...] * p


_tc_b = pl.pallas_call(
    _tc_b_body,
    grid=(N // BR,),
    in_specs=[
        pl.BlockSpec((NC, BR, D), lambda i: (0, i, 0)),
        pl.BlockSpec((BR, D), lambda i: (i, 0)),
        pl.BlockSpec((BR, 1), lambda i: (i, 0)),
        pl.BlockSpec((1, D), lambda i: (0, 0)),
        pl.BlockSpec((D, 1), lambda i: (0, 0)),
    ],
    out_specs=pl.BlockSpec((BR, 1), lambda i: (i, 0)),
    out_shape=jax.ShapeDtypeStruct((N, 1), jnp.float32),
)


def kernel(x, edge_index, W1, b1, W2, b2):
    pad_row = jnp.arange(PAD, dtype=jnp.int32) % N
    pad_col = N + jnp.arange(PAD, dtype=jnp.int32) % (NP - N)
    row3 = jnp.concatenate([edge_index[0], pad_row]).reshape(NW, NCHUNK, CK)
    col3 = jnp.concatenate([edge_index[1], pad_col]).reshape(NW, NCHUNK, CK)
    h = _tc_h(x, W1)
    degp = _sc_deg(col3)
    d0 = degp[0, :N, None]
    d1 = degp[1, :N, None]
    g, dis = _tc_a(h, d0, d1)
    aggp = _sc_agg1(g, row3, col3)
    q = _tc_b(aggp, g, dis, b1.reshape(1, D), W2)
    qp = jnp.pad(q.reshape(N), (0, NP - N))
    disp = jnp.pad(dis.reshape(N), (0, NP - N))
    b2v = jnp.broadcast_to(b2, (16,))
    outp = _sc_agg2(qp, row3, col3, disp, b2v)
    return (outp[0, :N] + outp[1, :N])[:, None]


# speedup vs baseline: 1.0240x; 1.0240x over previous
import functools

import jax
import jax.numpy as jnp
from jax import lax
from jax.experimental import pallas as pl
from jax.experimental.pallas import tpu as pltpu
from jax.experimental.pallas import tpu_sc as plsc

N = 10000
E = 320000
D = 128
NC = 2
NS = 16
NW = NC * NS
CK = 128
EP = 327680
PAD = EP - E
EPW = EP // NW
NCHUNK = EPW // CK
HCHUNK = NCHUNK // 2
NP = 10240
SLICE1 = NP // NS
ROWS_PT = NP // NS
ZROWS = 128
BR = 2000

_MESH = plsc.VectorSubcoreMesh(core_axis_name="c", subcore_axis_name="s")


@functools.partial(
    pl.kernel,
    out_type=jax.ShapeDtypeStruct((NC, NP), jnp.float32),
    mesh=_MESH,
    compiler_params=pltpu.CompilerParams(needs_layout_passes=False),
    scratch_types=[
        pltpu.VMEM((NCHUNK, CK), jnp.int32),
        pltpu.VMEM((CK,), jnp.float32),
        pltpu.VMEM((SLICE1,), jnp.float32),
        pltpu.VMEM_SHARED((NP,), jnp.float32),
        pltpu.SemaphoreType.DMA,
    ],
)
def _sc_deg(col3_hbm, deg_hbm, col_v, ones_v, zb_v, acc_sh, dsem):
    cid = lax.axis_index("c")
    sid = lax.axis_index("s")
    wid = cid * NS + sid
    for j in range(SLICE1 // 16):
        zb_v[pl.ds(j * 16, 16)] = jnp.zeros((16,), jnp.float32)
    for j in range(CK // 16):
        ones_v[pl.ds(j * 16, 16)] = jnp.ones((16,), jnp.float32)
    off = pl.multiple_of(sid * SLICE1, 8)
    pltpu.sync_copy(zb_v, acc_sh.at[pl.ds(off, SLICE1)])
    plsc.subcore_barrier()
    pltpu.sync_copy(col3_hbm.at[wid], col_v)

    def adesc(i):
        return pltpu.make_async_copy(ones_v, acc_sh.at[col_v.at[i]], dsem)

    def body(i, carry):
        a = i * 4
        for k in range(4):
            adesc(a + k).start(add=True)
        for k in range(4):
            adesc(a + k).wait()
        return carry

    lax.fori_loop(0, NCHUNK // 4, body, 0)
    plsc.subcore_barrier()
    pltpu.sync_copy(acc_sh.at[pl.ds(off, SLICE1)],
                    deg_hbm.at[cid, pl.ds(off, SLICE1)])


@functools.partial(
    pl.kernel,
    out_type=jax.ShapeDtypeStruct((NC, NP, D), jnp.float32),
    mesh=_MESH,
    compiler_params=pltpu.CompilerParams(needs_layout_passes=False),
    scratch_types=[
        pltpu.VMEM((HCHUNK, CK), jnp.int32),
        pltpu.VMEM((HCHUNK, CK), jnp.int32),
        pltpu.VMEM((CK, D), jnp.float32),
        pltpu.VMEM((CK, D), jnp.float32),
        pltpu.VMEM_SHARED((NP, D), jnp.float32),
        pltpu.SemaphoreType.DMA,
        pltpu.SemaphoreType.DMA,
    ],
)
def _sc_agg1(g_hbm, row3_hbm, col3_hbm, agg_hbm,
             row_v, col_v, rbuf0, rbuf1, acc_sh, gsem, ssem):
    cid = lax.axis_index("c")
    sid = lax.axis_index("s")
    wid = cid * NS + sid

    def zbody(r, carry):
        for c in range(D // 16):
            rbuf0[r, pl.ds(c * 16, 16)] = jnp.zeros((16,), jnp.float32)
        return carry

    lax.fori_loop(0, CK, zbody, 0)
    base = pl.multiple_of(sid * ROWS_PT, 8)
    for j in range(ROWS_PT // CK):
        pltpu.sync_copy(rbuf0, acc_sh.at[pl.ds(base + j * CK, CK)])
    plsc.subcore_barrier()

    def gdesc(i, buf):
        return pltpu.make_async_copy(g_hbm.at[row_v.at[i]], buf, gsem)

    def sdesc(i, buf):
        return pltpu.make_async_copy(buf, acc_sh.at[col_v.at[i]], ssem)

    for half in range(2):
        pltpu.sync_copy(row3_hbm.at[wid, pl.ds(half * HCHUNK, HCHUNK)], row_v)
        pltpu.sync_copy(col3_hbm.at[wid, pl.ds(half * HCHUNK, HCHUNK)], col_v)
        gdesc(0, rbuf0).start()
        gdesc(1, rbuf1).start()

        def body(i, carry):
            a = i * 2
            gdesc(a, rbuf0).wait()
            sdesc(a, rbuf0).start(add=True)
            gdesc(a + 1, rbuf1).wait()
            sdesc(a, rbuf0).wait()

            @pl.when(a + 2 < HCHUNK)
            def _():
                gdesc(a + 2, rbuf0).start()

            sdesc(a + 1, rbuf1).start(add=True)
            sdesc(a + 1, rbuf1).wait()

            @pl.when(a + 3 < HCHUNK)
            def _():
                gdesc(a + 3, rbuf1).start()

            return carry

        lax.fori_loop(0, HCHUNK // 2, body, 0)
    plsc.subcore_barrier()
    pltpu.sync_copy(acc_sh.at[pl.ds(base, ROWS_PT)],
                    agg_hbm.at[cid, pl.ds(base, ROWS_PT)])


@functools.partial(
    pl.kernel,
    out_type=jax.ShapeDtypeStruct((NC, NP), jnp.float32),
    mesh=_MESH,
    compiler_params=pltpu.CompilerParams(needs_layout_passes=False),
    scratch_types=[
        pltpu.VMEM((NCHUNK, CK), jnp.int32),
        pltpu.VMEM((NCHUNK, CK), jnp.int32),
        pltpu.VMEM((N,), jnp.float32),
        pltpu.VMEM((16,), jnp.float32),
        pltpu.VMEM((CK,), jnp.float32),
        pltpu.VMEM((CK,), jnp.float32),
        pltpu.VMEM((SLICE1,), jnp.float32),
        pltpu.VMEM((2 * SLICE1,), jnp.float32),
        pltpu.VMEM_SHARED((NP,), jnp.float32),
        pltpu.SemaphoreType.DMA,
    ],
)
def _sc_agg2(q_hbm, row3_hbm, col3_hbm, dis_hbm, b2_hbm, out_hbm,
             row_v, col_v, q_v, b2_v, vals0, vals1, zb_v, dq_v, acc_sh, asem):
    cid = lax.axis_index("c")
    sid = lax.axis_index("s")
    wid = cid * NS + sid
    for j in range(SLICE1 // 16):
        zb_v[pl.ds(j * 16, 16)] = jnp.zeros((16,), jnp.float32)
    off = pl.multiple_of(sid * SLICE1, 8)
    pltpu.sync_copy(zb_v, acc_sh.at[pl.ds(off, SLICE1)])
    plsc.subcore_barrier()
    pltpu.sync_copy(q_hbm.at[pl.ds(0, N)], q_v)
    pltpu.sync_copy(row3_hbm.at[wid], row_v)
    pltpu.sync_copy(col3_hbm.at[wid], col_v)
    pltpu.sync_copy(b2_hbm, b2_v)

    def fill(i, buf):
        for j in range(CK // 16):
            idx16 = row_v[i, pl.ds(j * 16, 16)]
            buf[pl.ds(j * 16, 16)] = plsc.load_gather(q_v, [idx16])

    def adesc(i, buf):
        return pltpu.make_async_copy(buf, acc_sh.at[col_v.at[i]], asem)

    def body(i, carry):
        a = i * 2
        fill(a, vals0)
        adesc(a, vals0).start(add=True)
        fill(a + 1, vals1)
        adesc(a + 1, vals1).start(add=True)
        adesc(a, vals0).wait()
        adesc(a + 1, vals1).wait()
        return carry

    lax.fori_loop(0, NCHUNK // 2, body, 0)
    plsc.subcore_barrier()
    pltpu.sync_copy(dis_hbm.at[pl.ds(off, SLICE1)], dq_v.at[pl.ds(0, SLICE1)])
    pltpu.sync_copy(q_hbm.at[pl.ds(off, SLICE1)], dq_v.at[pl.ds(SLICE1, SLICE1)])
    pltpu.sync_copy(acc_sh.at[pl.ds(off, SLICE1)], zb_v)
    for j in range(SLICE1 // 16):
        d16 = dq_v[pl.ds(j * 16, 16)]
        q16 = dq_v[pl.ds(SLICE1 + j * 16, 16)]
        a16 = zb_v[pl.ds(j * 16, 16)]
        zb_v[pl.ds(j * 16, 16)] = d16 * (a16 + 0.5 * q16) + 0.5 * b2_v[...]
    pltpu.sync_copy(zb_v, out_hbm.at[cid, pl.ds(off, SLICE1)])


def _tc_a_body(x_ref, w1_ref, d0_ref, d1_ref, g_ref, dis_ref):
    deg = d0_ref[...] + d1_ref[...] + 1.0
    dis = lax.rsqrt(deg)
    h = jnp.dot(x_ref[...], w1_ref[...], preferred_element_type=jnp.float32)
    g_ref[...] = h * dis
    dis_ref[...] = dis


_tc_a = pl.pallas_call(
    _tc_a_body,
    grid=(N // BR,),
    in_specs=[
        pl.BlockSpec((BR, D), lambda i: (i, 0)),
        pl.BlockSpec((D, D), lambda i: (0, 0)),
        pl.BlockSpec((BR, 1), lambda i: (i, 0)),
        pl.BlockSpec((BR, 1), lambda i: (i, 0)),
    ],
    out_specs=[
        pl.BlockSpec((BR, D), lambda i: (i, 0)),
        pl.BlockSpec((BR, 1), lambda i: (i, 0)),
    ],
    out_shape=[
        jax.ShapeDtypeStruct((N, D), jnp.float32),
        jax.ShapeDtypeStruct((N, 1), jnp.float32),
    ],
)


def _tc_b_body(ap_ref, g_ref, dis_ref, b1_ref, w2_ref, q_ref):
    s = ap_ref[0] + ap_ref[1] + g_ref[...]
    out1 = dis_ref[...] * s + b1_ref[...]
    h1 = jnp.maximum(out1, 0.0)
    p = jnp.dot(h1, w2_ref[...], preferred_element_type=jnp.float32)
    q_ref[...] = dis_ref[...] * p


_tc_b = pl.pallas_call(
    _tc_b_body,
    grid=(N // BR,),
    in_specs=[
        pl.BlockSpec((NC, BR, D), lambda i: (0, i, 0)),
        pl.BlockSpec((BR, D), lambda i: (i, 0)),
        pl.BlockSpec((BR, 1), lambda i: (i, 0)),
        pl.BlockSpec((1, D), lambda i: (0, 0)),
        pl.BlockSpec((D, 1), lambda i: (0, 0)),
    ],
    out_specs=pl.BlockSpec((BR, 1), lambda i: (i, 0)),
    out_shape=jax.ShapeDtypeStruct((N, 1), jnp.float32),
)


def kernel(x, edge_index, W1, b1, W2, b2):
    pad_row = jnp.arange(PAD, dtype=jnp.int32) % N
    pad_col = N + jnp.arange(PAD, dtype=jnp.int32) % (NP - N)
    row3 = jnp.concatenate([edge_index[0], pad_row]).reshape(NW, NCHUNK, CK)
    col3 = jnp.concatenate([edge_index[1], pad_col]).reshape(NW, NCHUNK, CK)
    degp = _sc_deg(col3)
    d0 = degp[0, :N, None]
    d1 = degp[1, :N, None]
    g, dis = _tc_a(x, W1, d0, d1)
    aggp = _sc_agg1(g, row3, col3)
    q = _tc_b(aggp, g, dis, b1.reshape(1, D), W2)
    qp = jnp.pad(q.reshape(N), (0, NP - N))
    disp = jnp.pad(dis.reshape(N), (0, NP - N))
    b2v = jnp.broadcast_to(b2, (16,))
    outp = _sc_agg2(qp, row3, col3, disp, b2v)
    return (outp[0, :N] + outp[1, :N])[:, None]


# speedup vs baseline: 1.0330x; 1.0087x over previous
import functools

import jax
import jax.numpy as jnp
from jax import lax
from jax.experimental import pallas as pl
from jax.experimental.pallas import tpu as pltpu
from jax.experimental.pallas import tpu_sc as plsc

N = 10000
E = 320000
D = 128
NC = 2
NS = 16
NW = NC * NS
CK = 128
EP = 327680
PAD = EP - E
EPW = EP // NW
NCHUNK = EPW // CK
HCHUNK = NCHUNK // 2
NP = 10240
SLICE1 = NP // NS
ROWS_PT = NP // NS
ZROWS = 128
BR = 5000

_MESH = plsc.VectorSubcoreMesh(core_axis_name="c", subcore_axis_name="s")


@functools.partial(
    pl.kernel,
    out_type=jax.ShapeDtypeStruct((NC, NP), jnp.float32),
    mesh=_MESH,
    compiler_params=pltpu.CompilerParams(needs_layout_passes=False),
    scratch_types=[
        pltpu.VMEM((NCHUNK, CK), jnp.int32),
        pltpu.VMEM((CK,), jnp.float32),
        pltpu.VMEM((SLICE1,), jnp.float32),
        pltpu.VMEM_SHARED((NP,), jnp.float32),
        pltpu.SemaphoreType.DMA,
    ],
)
def _sc_deg(col3_hbm, deg_hbm, col_v, ones_v, zb_v, acc_sh, dsem):
    cid = lax.axis_index("c")
    sid = lax.axis_index("s")
    wid = cid * NS + sid
    for j in range(SLICE1 // 16):
        zb_v[pl.ds(j * 16, 16)] = jnp.zeros((16,), jnp.float32)
    for j in range(CK // 16):
        ones_v[pl.ds(j * 16, 16)] = jnp.ones((16,), jnp.float32)
    off = pl.multiple_of(sid * SLICE1, 8)
    pltpu.sync_copy(zb_v, acc_sh.at[pl.ds(off, SLICE1)])
    plsc.subcore_barrier()
    pltpu.sync_copy(col3_hbm.at[wid], col_v)

    def adesc(i):
        return pltpu.make_async_copy(ones_v, acc_sh.at[col_v.at[i]], dsem)

    def body(i, carry):
        a = i * 4
        for k in range(4):
            adesc(a + k).start(add=True)
        for k in range(4):
            adesc(a + k).wait()
        return carry

    lax.fori_loop(0, NCHUNK // 4, body, 0)
    plsc.subcore_barrier()
    pltpu.sync_copy(acc_sh.at[pl.ds(off, SLICE1)],
                    deg_hbm.at[cid, pl.ds(off, SLICE1)])


@functools.partial(
    pl.kernel,
    out_type=jax.ShapeDtypeStruct((NC, NP, D), jnp.float32),
    mesh=_MESH,
    compiler_params=pltpu.CompilerParams(needs_layout_passes=False),
    scratch_types=[
        pltpu.VMEM((HCHUNK, CK), jnp.int32),
        pltpu.VMEM((HCHUNK, CK), jnp.int32),
        pltpu.VMEM((CK, D), jnp.float32),
        pltpu.VMEM((CK, D), jnp.float32),
        pltpu.VMEM_SHARED((NP, D), jnp.float32),
        pltpu.SemaphoreType.DMA,
        pltpu.SemaphoreType.DMA,
    ],
)
def _sc_agg1(g_hbm, row3_hbm, col3_hbm, agg_hbm,
             row_v, col_v, rbuf0, rbuf1, acc_sh, gsem, ssem):
    cid = lax.axis_index("c")
    sid = lax.axis_index("s")
    wid = cid * NS + sid

    def zbody(r, carry):
        for c in range(D // 16):
            rbuf0[r, pl.ds(c * 16, 16)] = jnp.zeros((16,), jnp.float32)
        return carry

    lax.fori_loop(0, CK, zbody, 0)
    base = pl.multiple_of(sid * ROWS_PT, 8)
    for j in range(ROWS_PT // CK):
        pltpu.sync_copy(rbuf0, acc_sh.at[pl.ds(base + j * CK, CK)])
    plsc.subcore_barrier()

    def gdesc(i, buf):
        return pltpu.make_async_copy(g_hbm.at[row_v.at[i]], buf, gsem)

    def sdesc(i, buf):
        return pltpu.make_async_copy(buf, acc_sh.at[col_v.at[i]], ssem)

    for half in range(2):
        pltpu.sync_copy(row3_hbm.at[wid, pl.ds(half * HCHUNK, HCHUNK)], row_v)
        pltpu.sync_copy(col3_hbm.at[wid, pl.ds(half * HCHUNK, HCHUNK)], col_v)
        gdesc(0, rbuf0).start()
        gdesc(1, rbuf1).start()

        def body(i, carry):
            a = i * 2
            gdesc(a, rbuf0).wait()
            sdesc(a, rbuf0).start(add=True)
            gdesc(a + 1, rbuf1).wait()
            sdesc(a, rbuf0).wait()

            @pl.when(a + 2 < HCHUNK)
            def _():
                gdesc(a + 2, rbuf0).start()

            sdesc(a + 1, rbuf1).start(add=True)
            sdesc(a + 1, rbuf1).wait()

            @pl.when(a + 3 < HCHUNK)
            def _():
                gdesc(a + 3, rbuf1).start()

            return carry

        lax.fori_loop(0, HCHUNK // 2, body, 0)
    plsc.subcore_barrier()
    pltpu.sync_copy(acc_sh.at[pl.ds(base, ROWS_PT)],
                    agg_hbm.at[cid, pl.ds(base, ROWS_PT)])


@functools.partial(
    pl.kernel,
    out_type=jax.ShapeDtypeStruct((NC, NP), jnp.float32),
    mesh=_MESH,
    compiler_params=pltpu.CompilerParams(needs_layout_passes=False),
    scratch_types=[
        pltpu.VMEM((NCHUNK, CK), jnp.int32),
        pltpu.VMEM((NCHUNK, CK), jnp.int32),
        pltpu.VMEM((N,), jnp.float32),
        pltpu.VMEM((16,), jnp.float32),
        pltpu.VMEM((CK,), jnp.float32),
        pltpu.VMEM((CK,), jnp.float32),
        pltpu.VMEM((SLICE1,), jnp.float32),
        pltpu.VMEM((2 * SLICE1,), jnp.float32),
        pltpu.VMEM_SHARED((NP,), jnp.float32),
        pltpu.SemaphoreType.DMA,
    ],
)
def _sc_agg2(q_hbm, row3_hbm, col3_hbm, dis_hbm, b2_hbm, out_hbm,
             row_v, col_v, q_v, b2_v, vals0, vals1, zb_v, dq_v, acc_sh, asem):
    cid = lax.axis_index("c")
    sid = lax.axis_index("s")
    wid = cid * NS + sid
    for j in range(SLICE1 // 16):
        zb_v[pl.ds(j * 16, 16)] = jnp.zeros((16,), jnp.float32)
    off = pl.multiple_of(sid * SLICE1, 8)
    pltpu.sync_copy(zb_v, acc_sh.at[pl.ds(off, SLICE1)])
    plsc.subcore_barrier()
    pltpu.sync_copy(q_hbm.at[pl.ds(0, N)], q_v)
    pltpu.sync_copy(row3_hbm.at[wid], row_v)
    pltpu.sync_copy(col3_hbm.at[wid], col_v)
    pltpu.sync_copy(b2_hbm, b2_v)

    def fill(i, buf):
        for j in range(CK // 16):
            idx16 = row_v[i, pl.ds(j * 16, 16)]
            buf[pl.ds(j * 16, 16)] = plsc.load_gather(q_v, [idx16])

    def adesc(i, buf):
        return pltpu.make_async_copy(buf, acc_sh.at[col_v.at[i]], asem)

    def body(i, carry):
        a = i * 2
        fill(a, vals0)
        adesc(a, vals0).start(add=True)
        fill(a + 1, vals1)
        adesc(a + 1, vals1).start(add=True)
        adesc(a, vals0).wait()
        adesc(a + 1, vals1).wait()
        return carry

    lax.fori_loop(0, NCHUNK // 2, body, 0)
    plsc.subcore_barrier()
    pltpu.sync_copy(dis_hbm.at[pl.ds(off, SLICE1)], dq_v.at[pl.ds(0, SLICE1)])
    pltpu.sync_copy(q_hbm.at[pl.ds(off, SLICE1)], dq_v.at[pl.ds(SLICE1, SLICE1)])
    pltpu.sync_copy(acc_sh.at[pl.ds(off, SLICE1)], zb_v)
    for j in range(SLICE1 // 16):
        d16 = dq_v[pl.ds(j * 16, 16)]
        q16 = dq_v[pl.ds(SLICE1 + j * 16, 16)]
        a16 = zb_v[pl.ds(j * 16, 16)]
        zb_v[pl.ds(j * 16, 16)] = d16 * (a16 + 0.5 * q16) + 0.5 * b2_v[...]
    pltpu.sync_copy(zb_v, out_hbm.at[cid, pl.ds(off, SLICE1)])


def _tc_a_body(x_ref, w1_ref, d0_ref, d1_ref, g_ref, dis_ref):
    deg = d0_ref[...] + d1_ref[...] + 1.0
    dis = lax.rsqrt(deg)
    h = jnp.dot(x_ref[...], w1_ref[...], preferred_element_type=jnp.float32)
    g_ref[...] = h * dis
    dis_ref[...] = dis


_tc_a = pl.pallas_call(
    _tc_a_body,
    grid=(N // BR,),
    in_specs=[
        pl.BlockSpec((BR, D), lambda i: (i, 0)),
        pl.BlockSpec((D, D), lambda i: (0, 0)),
        pl.BlockSpec((BR, 1), lambda i: (i, 0)),
        pl.BlockSpec((BR, 1), lambda i: (i, 0)),
    ],
    out_specs=[
        pl.BlockSpec((BR, D), lambda i: (i, 0)),
        pl.BlockSpec((BR, 1), lambda i: (i, 0)),
    ],
    out_shape=[
        jax.ShapeDtypeStruct((N, D), jnp.float32),
        jax.ShapeDtypeStruct((N, 1), jnp.float32),
    ],
)


def _tc_b_body(ap_ref, g_ref, dis_ref, b1_ref, w2_ref, q_ref):
    s = ap_ref[0] + ap_ref[1] + g_ref[...]
    out1 = dis_ref[...] * s + b1_ref[...]
    h1 = jnp.maximum(out1, 0.0)
    p = jnp.dot(h1, w2_ref[...], preferred_element_type=jnp.float32)
    q_ref[...] = dis_ref[...] * p


_tc_b = pl.pallas_call(
    _tc_b_body,
    grid=(N // BR,),
    in_specs=[
        pl.BlockSpec((NC, BR, D), lambda i: (0, i, 0)),
        pl.BlockSpec((BR, D), lambda i: (i, 0)),
        pl.BlockSpec((BR, 1), lambda i: (i, 0)),
        pl.BlockSpec((1, D), lambda i: (0, 0)),
        pl.BlockSpec((D, 1), lambda i: (0, 0)),
    ],
    out_specs=pl.BlockSpec((BR, 1), lambda i: (i, 0)),
    out_shape=jax.ShapeDtypeStruct((N, 1), jnp.float32),
)


def kernel(x, edge_index, W1, b1, W2, b2):
    pad_row = jnp.arange(PAD, dtype=jnp.int32) % N
    pad_col = N + jnp.arange(PAD, dtype=jnp.int32) % (NP - N)
    row3 = jnp.concatenate([edge_index[0], pad_row]).reshape(NW, NCHUNK, CK)
    col3 = jnp.concatenate([edge_index[1], pad_col]).reshape(NW, NCHUNK, CK)
    degp = _sc_deg(col3)
    d0 = degp[0, :N, None]
    d1 = degp[1, :N, None]
    g, dis = _tc_a(x, W1, d0, d1)
    aggp = _sc_agg1(g, row3, col3)
    q = _tc_b(aggp, g, dis, b1.reshape(1, D), W2)
    qp = jnp.pad(q.reshape(N), (0, NP - N))
    disp = jnp.pad(dis.reshape(N), (0, NP - N))
    b2v = jnp.broadcast_to(b2, (16,))
    outp = _sc_agg2(qp, row3, col3, disp, b2v)
    return (outp[0, :N] + outp[1, :N])[:, None]
